# baseline (speedup 1.0000x reference)
import functools

import jax
import jax.numpy as jnp
from jax import lax
from jax.experimental import pallas as pl
from jax.experimental.pallas import tpu as pltpu
from jax.experimental.pallas import tpu_sc as plsc

BATCH = 4096
SEQ = 200
SEQ_PAD = 208
HALF = SEQ_PAD // 2
EMB = 32
HID = 128
OUT = 64
VOCAB1 = 1000001
ROW_B = 4 * EMB

NUM_WORKERS = 32
ROWS_PER_W = BATCH // NUM_WORKERS
HALVES_PER_W = 2 * ROWS_PER_W
NBUF = 8
GROUPS = HALVES_PER_W // NBUF


def _pool_body(table_hbm, idx_hbm, out_hbm, idx_v, b0, b1, b2, b3, b4, b5,
               b6, b7, outs_v, s0, s1, s2, s3, s4, s5, s6, s7):
    bufs = (b0, b1, b2, b3, b4, b5, b6, b7)
    sems = (s0, s1, s2, s3, s4, s5, s6, s7)
    wid = lax.axis_index("s") * 2 + lax.axis_index("c")
    base_half = wid * HALVES_PER_W
    base_row = wid * ROWS_PER_W

    pltpu.sync_copy(idx_hbm.at[pl.ds(base_half, HALVES_PER_W)], idx_v)

    def group(g, carry):
        cps = [
            pltpu.async_copy(table_hbm.at[idx_v.at[NBUF * g + k]],
                             bufs[k], sems[k])
            for k in range(NBUF)
        ]
        for r in range(NBUF // 2):
            acc_lo = jnp.zeros((16,), jnp.float32)
            acc_hi = jnp.zeros((16,), jnp.float32)
            for k in (2 * r, 2 * r + 1):
                cps[k].wait()
                buf = bufs[k]
                for s in range(HALF):
                    acc_lo = acc_lo + plsc.bitcast(buf[s, 0:64],
                                                   jnp.float32)
                    acc_hi = acc_hi + plsc.bitcast(buf[s, 64:128],
                                                   jnp.float32)
            row = (NBUF // 2) * g + r
            outs_v[row, 0:16] = acc_lo
            outs_v[row, 16:32] = acc_hi
        return carry

    lax.fori_loop(0, GROUPS, group, 0)
    pltpu.sync_copy(outs_v, out_hbm.at[pl.ds(base_row, ROWS_PER_W)])


_pooled_sum = functools.partial(
    pl.kernel,
    mesh=plsc.VectorSubcoreMesh(core_axis_name="c", subcore_axis_name="s"),
    compiler_params=pltpu.CompilerParams(use_tc_tiling_on_sc=False,
                                         needs_layout_passes=False),
    out_type=jax.ShapeDtypeStruct((BATCH, EMB), jnp.float32),
    scratch_types=[
        pltpu.VMEM((HALVES_PER_W, HALF), jnp.int32),
        pltpu.VMEM((HALF, ROW_B), jnp.int8),
        pltpu.VMEM((HALF, ROW_B), jnp.int8),
        pltpu.VMEM((HALF, ROW_B), jnp.int8),
        pltpu.VMEM((HALF, ROW_B), jnp.int8),
        pltpu.VMEM((HALF, ROW_B), jnp.int8),
        pltpu.VMEM((HALF, ROW_B), jnp.int8),
        pltpu.VMEM((HALF, ROW_B), jnp.int8),
        pltpu.VMEM((HALF, ROW_B), jnp.int8),
        pltpu.VMEM((ROWS_PER_W, EMB), jnp.float32),
        pltpu.SemaphoreType.DMA,
        pltpu.SemaphoreType.DMA,
        pltpu.SemaphoreType.DMA,
        pltpu.SemaphoreType.DMA,
        pltpu.SemaphoreType.DMA,
        pltpu.SemaphoreType.DMA,
        pltpu.SemaphoreType.DMA,
        pltpu.SemaphoreType.DMA,
    ],
)(_pool_body)


def _mlp_body(pooled_ref, idx_ref, t0_ref, w1_ref, bb1_ref, w2_ref, bb2_ref,
              out_ref):
    pooled = pooled_ref[...]
    idx = idx_ref[...]
    c0 = jnp.sum((idx == 0).astype(jnp.float32), axis=1, keepdims=True) + 8.0
    x = (pooled - c0 * t0_ref[...]) * (1.0 / SEQ)
    h = jnp.dot(x, w1_ref[...], preferred_element_type=jnp.float32,
                precision=lax.Precision.HIGHEST) + bb1_ref[...]
    z = jnp.dot(h, w2_ref[...], preferred_element_type=jnp.float32,
                precision=lax.Precision.HIGHEST) + bb2_ref[...]
    z = z - jnp.max(z, axis=1, keepdims=True)
    e = jnp.exp(z)
    out_ref[...] = e / jnp.sum(e, axis=1, keepdims=True)


def _mlp_call(pooled, idx, t0, w1, bb1, w2, bb2):
    bt = 512
    grid = (BATCH // bt,)
    return pl.pallas_call(
        _mlp_body,
        out_shape=jax.ShapeDtypeStruct((BATCH, OUT), jnp.float32),
        grid=grid,
        in_specs=[
            pl.BlockSpec((bt, EMB), lambda i: (i, 0)),
            pl.BlockSpec((bt, SEQ), lambda i: (i, 0)),
            pl.BlockSpec((1, EMB), lambda i: (0, 0)),
            pl.BlockSpec((EMB, HID), lambda i: (0, 0)),
            pl.BlockSpec((1, HID), lambda i: (0, 0)),
            pl.BlockSpec((HID, OUT), lambda i: (0, 0)),
            pl.BlockSpec((1, OUT), lambda i: (0, 0)),
        ],
        out_specs=pl.BlockSpec((bt, OUT), lambda i: (i, 0)),
    )(pooled, idx, t0, w1, bb1, w2, bb2)


def kernel(inputs, table, W1, b1, W2, b2):
    idx = inputs.astype(jnp.int32)
    idx_pad = jnp.pad(idx, ((0, 0), (0, SEQ_PAD - SEQ)))
    idx_halves = idx_pad.reshape(BATCH * 2, HALF)
    table_b = jax.lax.bitcast_convert_type(table, jnp.int8)
    table_b = table_b.reshape(VOCAB1, ROW_B)
    pooled = _pooled_sum(table_b, idx_halves)
    t0 = table[0:1]
    return _mlp_call(pooled, idx, t0, W1, b1.reshape(1, HID), W2,
                     b2.reshape(1, OUT))


# speedup vs baseline: 2.0127x; 2.0127x over previous
import functools

import jax
import jax.numpy as jnp
from jax import lax
from jax.experimental import pallas as pl
from jax.experimental.pallas import tpu as pltpu
from jax.experimental.pallas import tpu_sc as plsc

BATCH = 4096
SEQ = 200
SEQ_PAD = 208
HALF = SEQ_PAD // 2
EMB = 32
HID = 128
OUT = 64
VOCAB1 = 1000001
PERM = tuple(range(0, EMB, 2)) + tuple(range(1, EMB, 2))

NUM_WORKERS = 32
ROWS_PER_W = BATCH // NUM_WORKERS
HALVES_PER_W = 2 * ROWS_PER_W
NBUF = 8
GROUPS = HALVES_PER_W // NBUF


def _pool_body(table_hbm, idx_hbm, out_hbm, idx_v, b0, b1, b2, b3, b4, b5,
               b6, b7, outs_v, s0, s1, s2, s3, s4, s5, s6, s7):
    bufs = (b0, b1, b2, b3, b4, b5, b6, b7)
    sems = (s0, s1, s2, s3, s4, s5, s6, s7)
    wid = lax.axis_index("s") * 2 + lax.axis_index("c")
    base_half = wid * HALVES_PER_W
    base_row = wid * ROWS_PER_W

    pltpu.sync_copy(idx_hbm.at[pl.ds(base_half, HALVES_PER_W)], idx_v)

    def group(g, carry):
        cps = [
            pltpu.async_copy(table_hbm.at[idx_v.at[NBUF * g + k]],
                             bufs[k], sems[k])
            for k in range(NBUF)
        ]
        for r in range(NBUF // 2):
            acc_lo = jnp.zeros((16,), jnp.float32)
            acc_hi = jnp.zeros((16,), jnp.float32)
            for k in (2 * r, 2 * r + 1):
                cps[k].wait()
                buf = bufs[k]
                for s in range(HALF):
                    ev, od = plsc.unpack(buf[s, 0:32],
                                         format=plsc.PackFormat.INTERLEAVED)
                    acc_lo = acc_lo + ev
                    acc_hi = acc_hi + od
            row = (NBUF // 2) * g + r
            outs_v[row, 0:16] = acc_lo
            outs_v[row, 16:32] = acc_hi
        return carry

    lax.fori_loop(0, GROUPS, group, 0)
    pltpu.sync_copy(outs_v, out_hbm.at[pl.ds(base_row, ROWS_PER_W)])


_pooled_sum = functools.partial(
    pl.kernel,
    mesh=plsc.VectorSubcoreMesh(core_axis_name="c", subcore_axis_name="s"),
    compiler_params=pltpu.CompilerParams(use_tc_tiling_on_sc=False,
                                         needs_layout_passes=False),
    out_type=jax.ShapeDtypeStruct((BATCH, EMB), jnp.float32),
    scratch_types=[
        pltpu.VMEM((HALVES_PER_W, HALF), jnp.int32),
        pltpu.VMEM((HALF, EMB), jnp.bfloat16),
        pltpu.VMEM((HALF, EMB), jnp.bfloat16),
        pltpu.VMEM((HALF, EMB), jnp.bfloat16),
        pltpu.VMEM((HALF, EMB), jnp.bfloat16),
        pltpu.VMEM((HALF, EMB), jnp.bfloat16),
        pltpu.VMEM((HALF, EMB), jnp.bfloat16),
        pltpu.VMEM((HALF, EMB), jnp.bfloat16),
        pltpu.VMEM((HALF, EMB), jnp.bfloat16),
        pltpu.VMEM((ROWS_PER_W, EMB), jnp.float32),
        pltpu.SemaphoreType.DMA,
        pltpu.SemaphoreType.DMA,
        pltpu.SemaphoreType.DMA,
        pltpu.SemaphoreType.DMA,
        pltpu.SemaphoreType.DMA,
        pltpu.SemaphoreType.DMA,
        pltpu.SemaphoreType.DMA,
        pltpu.SemaphoreType.DMA,
    ],
)(_pool_body)


def _cast_body(t_ref, o_ref):
    o_ref[...] = t_ref[...].astype(jnp.bfloat16)


def _cast_call(table):
    blk = 8192
    return pl.pallas_call(
        _cast_body,
        out_shape=jax.ShapeDtypeStruct((VOCAB1, EMB), jnp.bfloat16),
        grid=(pl.cdiv(VOCAB1, blk),),
        in_specs=[pl.BlockSpec((blk, EMB), lambda i: (i, 0))],
        out_specs=pl.BlockSpec((blk, EMB), lambda i: (i, 0)),
    )(table)


def _mlp_body(pooled_ref, idx_ref, t0_ref, w1_ref, bb1_ref, w2_ref, bb2_ref,
              out_ref):
    pooled = pooled_ref[...]
    idx = idx_ref[...]
    c0 = jnp.sum((idx == 0).astype(jnp.float32), axis=1, keepdims=True) + 8.0
    x = (pooled - c0 * t0_ref[...]) * (1.0 / SEQ)
    h = jnp.dot(x, w1_ref[...], preferred_element_type=jnp.float32,
                precision=lax.Precision.HIGHEST) + bb1_ref[...]
    z = jnp.dot(h, w2_ref[...], preferred_element_type=jnp.float32,
                precision=lax.Precision.HIGHEST) + bb2_ref[...]
    z = z - jnp.max(z, axis=1, keepdims=True)
    e = jnp.exp(z)
    out_ref[...] = e / jnp.sum(e, axis=1, keepdims=True)


def _mlp_call(pooled, idx, t0, w1, bb1, w2, bb2):
    bt = 512
    grid = (BATCH // bt,)
    return pl.pallas_call(
        _mlp_body,
        out_shape=jax.ShapeDtypeStruct((BATCH, OUT), jnp.float32),
        grid=grid,
        in_specs=[
            pl.BlockSpec((bt, EMB), lambda i: (i, 0)),
            pl.BlockSpec((bt, SEQ), lambda i: (i, 0)),
            pl.BlockSpec((1, EMB), lambda i: (0, 0)),
            pl.BlockSpec((EMB, HID), lambda i: (0, 0)),
            pl.BlockSpec((1, HID), lambda i: (0, 0)),
            pl.BlockSpec((HID, OUT), lambda i: (0, 0)),
            pl.BlockSpec((1, OUT), lambda i: (0, 0)),
        ],
        out_specs=pl.BlockSpec((bt, OUT), lambda i: (i, 0)),
    )(pooled, idx, t0, w1, bb1, w2, bb2)


def kernel(inputs, table, W1, b1, W2, b2):
    idx = inputs.astype(jnp.int32)
    idx_pad = jnp.pad(idx, ((0, 0), (0, SEQ_PAD - SEQ)))
    idx_halves = idx_pad.reshape(BATCH * 2, HALF)
    table_h = _cast_call(table)
    pooled = _pooled_sum(table_h, idx_halves)
    perm = jnp.array(PERM, jnp.int32)
    t0s = table_h[0:1].astype(jnp.float32)[:, perm]
    w1s = W1[perm, :]
    return _mlp_call(pooled, idx, t0s, w1s, b1.reshape(1, HID), W2,
                     b2.reshape(1, OUT))


# speedup vs baseline: 2.6349x; 1.3091x over previous
import functools

import jax
import jax.numpy as jnp
from jax import lax
from jax.experimental import pallas as pl
from jax.experimental.pallas import tpu as pltpu
from jax.experimental.pallas import tpu_sc as plsc

BATCH = 4096
SEQ = 200
SEQ_PAD = 208
HALF = SEQ_PAD // 2
EMB = 32
HID = 128
OUT = 64
VOCAB1 = 1000001
PERM = tuple(range(0, EMB, 2)) + tuple(range(1, EMB, 2))

NUM_WORKERS = 32
ROWS_PER_W = BATCH // NUM_WORKERS
HALVES_PER_W = 2 * ROWS_PER_W
NBUF = 8
GROUPS = HALVES_PER_W // NBUF


def _pool_body(table_hbm, idx_hbm, out_hbm, idx_v, b0, b1, b2, b3, b4, b5,
               b6, b7, outs_v, s0, s1, s2, s3, s4, s5, s6, s7):
    bufs = (b0, b1, b2, b3, b4, b5, b6, b7)
    sems = (s0, s1, s2, s3, s4, s5, s6, s7)
    wid = lax.axis_index("s") * 2 + lax.axis_index("c")
    base_half = wid * HALVES_PER_W
    base_row = wid * ROWS_PER_W

    pltpu.sync_copy(idx_hbm.at[pl.ds(base_half, HALVES_PER_W)], idx_v)

    def group(g, carry):
        cps = [
            pltpu.async_copy(table_hbm.at[idx_v.at[NBUF * g + k]],
                             bufs[k], sems[k])
            for k in range(NBUF)
        ]
        for r in range(NBUF // 2):
            acc_lo = jnp.zeros((16,), jnp.float32)
            acc_hi = jnp.zeros((16,), jnp.float32)
            for k in (2 * r, 2 * r + 1):
                cps[k].wait()
                buf = bufs[k]
                for s in range(HALF):
                    ev, od = plsc.unpack(buf[s, 0:32],
                                         format=plsc.PackFormat.INTERLEAVED)
                    acc_lo = acc_lo + ev
                    acc_hi = acc_hi + od
            row = (NBUF // 2) * g + r
            outs_v[row, 0:16] = acc_lo
            outs_v[row, 16:32] = acc_hi
        return carry

    lax.fori_loop(0, GROUPS, group, 0)
    pltpu.sync_copy(outs_v, out_hbm.at[pl.ds(base_row, ROWS_PER_W)])


_pooled_sum = functools.partial(
    pl.kernel,
    mesh=plsc.VectorSubcoreMesh(core_axis_name="c", subcore_axis_name="s"),
    compiler_params=pltpu.CompilerParams(use_tc_tiling_on_sc=False,
                                         needs_layout_passes=False),
    out_type=jax.ShapeDtypeStruct((BATCH, EMB), jnp.float32),
    scratch_types=[
        pltpu.VMEM((HALVES_PER_W, HALF), jnp.int32),
        pltpu.VMEM((HALF, EMB), jnp.bfloat16),
        pltpu.VMEM((HALF, EMB), jnp.bfloat16),
        pltpu.VMEM((HALF, EMB), jnp.bfloat16),
        pltpu.VMEM((HALF, EMB), jnp.bfloat16),
        pltpu.VMEM((HALF, EMB), jnp.bfloat16),
        pltpu.VMEM((HALF, EMB), jnp.bfloat16),
        pltpu.VMEM((HALF, EMB), jnp.bfloat16),
        pltpu.VMEM((HALF, EMB), jnp.bfloat16),
        pltpu.VMEM((ROWS_PER_W, EMB), jnp.float32),
        pltpu.SemaphoreType.DMA,
        pltpu.SemaphoreType.DMA,
        pltpu.SemaphoreType.DMA,
        pltpu.SemaphoreType.DMA,
        pltpu.SemaphoreType.DMA,
        pltpu.SemaphoreType.DMA,
        pltpu.SemaphoreType.DMA,
        pltpu.SemaphoreType.DMA,
    ],
)(_pool_body)


def _mlp_body(pooled_ref, idx_ref, t0_ref, w1_ref, bb1_ref, w2_ref, bb2_ref,
              out_ref):
    pooled = pooled_ref[...]
    idx = idx_ref[...]
    c0 = jnp.sum((idx == 0).astype(jnp.float32), axis=1, keepdims=True) + 8.0
    x = (pooled - c0 * t0_ref[...]) * (1.0 / SEQ)
    h = jnp.dot(x, w1_ref[...], preferred_element_type=jnp.float32,
                precision=lax.Precision.HIGHEST) + bb1_ref[...]
    z = jnp.dot(h, w2_ref[...], preferred_element_type=jnp.float32,
                precision=lax.Precision.HIGHEST) + bb2_ref[...]
    z = z - jnp.max(z, axis=1, keepdims=True)
    e = jnp.exp(z)
    out_ref[...] = e / jnp.sum(e, axis=1, keepdims=True)


def _mlp_call(pooled, idx, t0, w1, bb1, w2, bb2):
    bt = 512
    grid = (BATCH // bt,)
    return pl.pallas_call(
        _mlp_body,
        out_shape=jax.ShapeDtypeStruct((BATCH, OUT), jnp.float32),
        grid=grid,
        in_specs=[
            pl.BlockSpec((bt, EMB), lambda i: (i, 0)),
            pl.BlockSpec((bt, SEQ), lambda i: (i, 0)),
            pl.BlockSpec((1, EMB), lambda i: (0, 0)),
            pl.BlockSpec((EMB, HID), lambda i: (0, 0)),
            pl.BlockSpec((1, HID), lambda i: (0, 0)),
            pl.BlockSpec((HID, OUT), lambda i: (0, 0)),
            pl.BlockSpec((1, OUT), lambda i: (0, 0)),
        ],
        out_specs=pl.BlockSpec((bt, OUT), lambda i: (i, 0)),
    )(pooled, idx, t0, w1, bb1, w2, bb2)


def kernel(inputs, table, W1, b1, W2, b2):
    idx = inputs.astype(jnp.int32)
    idx_pad = jnp.pad(idx, ((0, 0), (0, SEQ_PAD - SEQ)))
    idx_halves = idx_pad.reshape(BATCH * 2, HALF)
    table_h = table.astype(jnp.bfloat16)
    pooled = _pooled_sum(table_h, idx_halves)
    perm = jnp.array(PERM, jnp.int32)
    t0s = table_h[0:1].astype(jnp.float32)[:, perm]
    w1s = W1[perm, :]
    return _mlp_call(pooled, idx, t0s, w1s, b1.reshape(1, HID), W2,
                     b2.reshape(1, OUT))


# speedup vs baseline: 2.9926x; 1.1357x over previous
import functools

import jax
import jax.numpy as jnp
from jax import lax
from jax.experimental import pallas as pl
from jax.experimental.pallas import tpu as pltpu
from jax.experimental.pallas import tpu_sc as plsc

BATCH = 4096
SEQ = 200
SEQ_PAD = 208
HALF = SEQ_PAD // 2
EMB = 32
HID = 128
OUT = 64

NUM_WORKERS = 32
ROWS_PER_W = BATCH // NUM_WORKERS
HALVES_PER_W = 2 * ROWS_PER_W
NBUF = 8
GROUPS = HALVES_PER_W // NBUF


def _pool_body(table_hbm, idx_hbm, out_hbm, idx_v, b0, b1, b2, b3, b4, b5,
               b6, b7, outs_v, s0, s1, s2, s3, s4, s5, s6, s7):
    bufs = (b0, b1, b2, b3, b4, b5, b6, b7)
    sems = (s0, s1, s2, s3, s4, s5, s6, s7)
    wid = lax.axis_index("s") * 2 + lax.axis_index("c")
    base_half = wid * HALVES_PER_W
    base_row = wid * ROWS_PER_W

    pltpu.sync_copy(idx_hbm.at[pl.ds(base_half, HALVES_PER_W)], idx_v)

    def group(g, carry):
        cps = [
            pltpu.async_copy(table_hbm.at[idx_v.at[NBUF * g + k]],
                             bufs[k], sems[k])
            for k in range(NBUF)
        ]
        for r in range(NBUF // 2):
            acc_lo = jnp.zeros((16,), jnp.float32)
            acc_hi = jnp.zeros((16,), jnp.float32)
            for k in (2 * r, 2 * r + 1):
                cps[k].wait()
                buf = bufs[k]
                for s in range(HALF):
                    acc_lo = acc_lo + buf[s, 0:16]
                    acc_hi = acc_hi + buf[s, 16:32]
            row = (NBUF // 2) * g + r
            outs_v[row, 0:16] = acc_lo
            outs_v[row, 16:32] = acc_hi
        return carry

    lax.fori_loop(0, GROUPS, group, 0)
    pltpu.sync_copy(outs_v, out_hbm.at[pl.ds(base_row, ROWS_PER_W)])


_pooled_sum = functools.partial(
    pl.kernel,
    mesh=plsc.VectorSubcoreMesh(core_axis_name="c", subcore_axis_name="s"),
    compiler_params=pltpu.CompilerParams(use_tc_tiling_on_sc=False),
    out_type=jax.ShapeDtypeStruct((BATCH, EMB), jnp.float32),
    scratch_types=[
        pltpu.VMEM((HALVES_PER_W, HALF), jnp.int32),
        pltpu.VMEM((HALF, EMB), jnp.float32),
        pltpu.VMEM((HALF, EMB), jnp.float32),
        pltpu.VMEM((HALF, EMB), jnp.float32),
        pltpu.VMEM((HALF, EMB), jnp.float32),
        pltpu.VMEM((HALF, EMB), jnp.float32),
        pltpu.VMEM((HALF, EMB), jnp.float32),
        pltpu.VMEM((HALF, EMB), jnp.float32),
        pltpu.VMEM((HALF, EMB), jnp.float32),
        pltpu.VMEM((ROWS_PER_W, EMB), jnp.float32),
        pltpu.SemaphoreType.DMA,
        pltpu.SemaphoreType.DMA,
        pltpu.SemaphoreType.DMA,
        pltpu.SemaphoreType.DMA,
        pltpu.SemaphoreType.DMA,
        pltpu.SemaphoreType.DMA,
        pltpu.SemaphoreType.DMA,
        pltpu.SemaphoreType.DMA,
    ],
)(_pool_body)


def _mlp_body(pooled_ref, idx_ref, t0_ref, w1_ref, bb1_ref, w2_ref, bb2_ref,
              out_ref):
    pooled = pooled_ref[...]
    idx = idx_ref[...]
    c0 = jnp.sum((idx == 0).astype(jnp.float32), axis=1, keepdims=True) + 8.0
    x = (pooled - c0 * t0_ref[...]) * (1.0 / SEQ)
    h = jnp.dot(x, w1_ref[...], preferred_element_type=jnp.float32,
                precision=lax.Precision.HIGHEST) + bb1_ref[...]
    z = jnp.dot(h, w2_ref[...], preferred_element_type=jnp.float32,
                precision=lax.Precision.HIGHEST) + bb2_ref[...]
    z = z - jnp.max(z, axis=1, keepdims=True)
    e = jnp.exp(z)
    out_ref[...] = e / jnp.sum(e, axis=1, keepdims=True)


def _mlp_call(pooled, idx, t0, w1, bb1, w2, bb2):
    bt = 512
    grid = (BATCH // bt,)
    return pl.pallas_call(
        _mlp_body,
        out_shape=jax.ShapeDtypeStruct((BATCH, OUT), jnp.float32),
        grid=grid,
        in_specs=[
            pl.BlockSpec((bt, EMB), lambda i: (i, 0)),
            pl.BlockSpec((bt, SEQ), lambda i: (i, 0)),
            pl.BlockSpec((1, EMB), lambda i: (0, 0)),
            pl.BlockSpec((EMB, HID), lambda i: (0, 0)),
            pl.BlockSpec((1, HID), lambda i: (0, 0)),
            pl.BlockSpec((HID, OUT), lambda i: (0, 0)),
            pl.BlockSpec((1, OUT), lambda i: (0, 0)),
        ],
        out_specs=pl.BlockSpec((bt, OUT), lambda i: (i, 0)),
    )(pooled, idx, t0, w1, bb1, w2, bb2)


def kernel(inputs, table, W1, b1, W2, b2):
    idx = inputs.astype(jnp.int32)
    idx_pad = jnp.pad(idx, ((0, 0), (0, SEQ_PAD - SEQ)))
    idx_halves = idx_pad.reshape(BATCH * 2, HALF)
    pooled = _pooled_sum(table, idx_halves)
    t0 = table[0:1]
    return _mlp_call(pooled, idx, t0, W1, b1.reshape(1, HID), W2,
                     b2.reshape(1, OUT))


# speedup vs baseline: 4.1626x; 1.3910x over previous
import functools

import jax
import jax.numpy as jnp
from jax import lax
from jax.experimental import pallas as pl
from jax.experimental.pallas import tpu as pltpu
from jax.experimental.pallas import tpu_sc as plsc

BATCH = 4096
SEQ = 200
SLC0 = 104
SLC1 = SEQ - SLC0
EMB = 32
HID = 128
OUT = 64

NUM_WORKERS = 32
ROWS_PER_W = BATCH // NUM_WORKERS
ROWS_PER_G = 4
GROUPS = ROWS_PER_W // ROWS_PER_G


def _pool_body(table_hbm, idx_hbm, out_hbm, idx_v, b0, b1, b2, b3, b4, b5,
               b6, b7, outs_v, s0, s1, s2, s3, s4, s5, s6, s7):
    bufs = (b0, b1, b2, b3, b4, b5, b6, b7)
    sems = (s0, s1, s2, s3, s4, s5, s6, s7)
    wid = lax.axis_index("s") * 2 + lax.axis_index("c")
    base_row = wid * ROWS_PER_W

    pltpu.sync_copy(idx_hbm.at[pl.ds(base_row, ROWS_PER_W)], idx_v)

    def group(g, carry):
        cps = []
        for k in range(2 * ROWS_PER_G):
            row = ROWS_PER_G * g + k // 2
            sl = pl.ds(0, SLC0) if k % 2 == 0 else pl.ds(SLC0, SLC1)
            cps.append(pltpu.async_copy(table_hbm.at[idx_v.at[row, sl]],
                                        bufs[k], sems[k]))
        for r in range(ROWS_PER_G):
            acc_lo = jnp.zeros((16,), jnp.float32)
            acc_hi = jnp.zeros((16,), jnp.float32)
            for k in (2 * r, 2 * r + 1):
                cps[k].wait()
                buf = bufs[k]
                for s in range(SLC0 if k % 2 == 0 else SLC1):
                    acc_lo = acc_lo + buf[s, 0:16]
                    acc_hi = acc_hi + buf[s, 16:32]
            row = ROWS_PER_G * g + r
            outs_v[row, 0:16] = acc_lo
            outs_v[row, 16:32] = acc_hi
        return carry

    lax.fori_loop(0, GROUPS, group, 0)
    pltpu.sync_copy(outs_v, out_hbm.at[pl.ds(base_row, ROWS_PER_W)])


_pooled_sum = functools.partial(
    pl.kernel,
    mesh=plsc.VectorSubcoreMesh(core_axis_name="c", subcore_axis_name="s"),
    compiler_params=pltpu.CompilerParams(use_tc_tiling_on_sc=False),
    out_type=jax.ShapeDtypeStruct((BATCH, EMB), jnp.float32),
    scratch_types=[
        pltpu.VMEM((ROWS_PER_W, SEQ), jnp.int32),
        pltpu.VMEM((SLC0, EMB), jnp.float32),
        pltpu.VMEM((SLC1, EMB), jnp.float32),
        pltpu.VMEM((SLC0, EMB), jnp.float32),
        pltpu.VMEM((SLC1, EMB), jnp.float32),
        pltpu.VMEM((SLC0, EMB), jnp.float32),
        pltpu.VMEM((SLC1, EMB), jnp.float32),
        pltpu.VMEM((SLC0, EMB), jnp.float32),
        pltpu.VMEM((SLC1, EMB), jnp.float32),
        pltpu.VMEM((ROWS_PER_W, EMB), jnp.float32),
        pltpu.SemaphoreType.DMA,
        pltpu.SemaphoreType.DMA,
        pltpu.SemaphoreType.DMA,
        pltpu.SemaphoreType.DMA,
        pltpu.SemaphoreType.DMA,
        pltpu.SemaphoreType.DMA,
        pltpu.SemaphoreType.DMA,
        pltpu.SemaphoreType.DMA,
    ],
)(_pool_body)


def _mlp_body(pooled_ref, idx_ref, t0_ref, w1_ref, bb1_ref, w2_ref, bb2_ref,
              out_ref):
    pooled = pooled_ref[...]
    idx = idx_ref[...]
    c0 = jnp.sum((idx == 0).astype(jnp.float32), axis=1, keepdims=True)
    x = (pooled - c0 * t0_ref[...]) * (1.0 / SEQ)
    h = jnp.dot(x, w1_ref[...], preferred_element_type=jnp.float32,
                precision=lax.Precision.HIGHEST) + bb1_ref[...]
    z = jnp.dot(h, w2_ref[...], preferred_element_type=jnp.float32,
                precision=lax.Precision.HIGHEST) + bb2_ref[...]
    z = z - jnp.max(z, axis=1, keepdims=True)
    e = jnp.exp(z)
    out_ref[...] = e / jnp.sum(e, axis=1, keepdims=True)


def _mlp_call(pooled, idx, t0, w1, bb1, w2, bb2):
    bt = 512
    grid = (BATCH // bt,)
    return pl.pallas_call(
        _mlp_body,
        out_shape=jax.ShapeDtypeStruct((BATCH, OUT), jnp.float32),
        grid=grid,
        in_specs=[
            pl.BlockSpec((bt, EMB), lambda i: (i, 0)),
            pl.BlockSpec((bt, SEQ), lambda i: (i, 0)),
            pl.BlockSpec((1, EMB), lambda i: (0, 0)),
            pl.BlockSpec((EMB, HID), lambda i: (0, 0)),
            pl.BlockSpec((1, HID), lambda i: (0, 0)),
            pl.BlockSpec((HID, OUT), lambda i: (0, 0)),
            pl.BlockSpec((1, OUT), lambda i: (0, 0)),
        ],
        out_specs=pl.BlockSpec((bt, OUT), lambda i: (i, 0)),
    )(pooled, idx, t0, w1, bb1, w2, bb2)


def kernel(inputs, table, W1, b1, W2, b2):
    idx = inputs.astype(jnp.int32)
    pooled = _pooled_sum(table, idx)
    t0 = table[0:1]
    return _mlp_call(pooled, idx, t0, W1, b1.reshape(1, HID), W2,
                     b2.reshape(1, OUT))


# speedup vs baseline: 4.1896x; 1.0065x over previous
import functools

import jax
import jax.numpy as jnp
from jax import lax
from jax.experimental import pallas as pl
from jax.experimental.pallas import tpu as pltpu
from jax.experimental.pallas import tpu_sc as plsc

BATCH = 4096
SEQ = 200
SLC0 = 104
SLC1 = SEQ - SLC0
EMB = 32
HID = 128
OUT = 64

NUM_WORKERS = 32
ROWS_PER_W = BATCH // NUM_WORKERS
ROWS_PER_G = 8
GROUPS = ROWS_PER_W // ROWS_PER_G


def _pool_body(table_hbm, idx_hbm, out_hbm, idx_v, b0, b1, b2, b3, b4, b5,
               b6, b7, b8, b9, b10, b11, b12, b13, b14, b15, outs_v,
               s0, s1, s2, s3, s4, s5, s6, s7, s8, s9, s10, s11, s12, s13,
               s14, s15):
    bufs = (b0, b1, b2, b3, b4, b5, b6, b7, b8, b9, b10, b11, b12, b13,
            b14, b15)
    sems = (s0, s1, s2, s3, s4, s5, s6, s7, s8, s9, s10, s11, s12, s13,
            s14, s15)
    wid = lax.axis_index("s") * 2 + lax.axis_index("c")
    base_row = wid * ROWS_PER_W

    pltpu.sync_copy(idx_hbm.at[pl.ds(base_row, ROWS_PER_W)], idx_v)

    def group(g, carry):
        cps = []
        for k in range(2 * ROWS_PER_G):
            row = ROWS_PER_G * g + k // 2
            sl = pl.ds(0, SLC0) if k % 2 == 0 else pl.ds(SLC0, SLC1)
            cps.append(pltpu.async_copy(table_hbm.at[idx_v.at[row, sl]],
                                        bufs[k], sems[k]))
        for r in range(ROWS_PER_G):
            acc_lo = jnp.zeros((16,), jnp.float32)
            acc_hi = jnp.zeros((16,), jnp.float32)
            for k in (2 * r, 2 * r + 1):
                cps[k].wait()
                buf = bufs[k]
                for s in range(SLC0 if k % 2 == 0 else SLC1):
                    acc_lo = acc_lo + buf[s, 0:16]
                    acc_hi = acc_hi + buf[s, 16:32]
            row = ROWS_PER_G * g + r
            outs_v[row, 0:16] = acc_lo
            outs_v[row, 16:32] = acc_hi
        return carry

    lax.fori_loop(0, GROUPS, group, 0)
    pltpu.sync_copy(outs_v, out_hbm.at[pl.ds(base_row, ROWS_PER_W)])


_pooled_sum = functools.partial(
    pl.kernel,
    mesh=plsc.VectorSubcoreMesh(core_axis_name="c", subcore_axis_name="s"),
    compiler_params=pltpu.CompilerParams(use_tc_tiling_on_sc=False),
    out_type=jax.ShapeDtypeStruct((BATCH, EMB), jnp.float32),
    scratch_types=[
        pltpu.VMEM((ROWS_PER_W, SEQ), jnp.int32),
        pltpu.VMEM((SLC0, EMB), jnp.float32),
        pltpu.VMEM((SLC1, EMB), jnp.float32),
        pltpu.VMEM((SLC0, EMB), jnp.float32),
        pltpu.VMEM((SLC1, EMB), jnp.float32),
        pltpu.VMEM((SLC0, EMB), jnp.float32),
        pltpu.VMEM((SLC1, EMB), jnp.float32),
        pltpu.VMEM((SLC0, EMB), jnp.float32),
        pltpu.VMEM((SLC1, EMB), jnp.float32),
        pltpu.VMEM((SLC0, EMB), jnp.float32),
        pltpu.VMEM((SLC1, EMB), jnp.float32),
        pltpu.VMEM((SLC0, EMB), jnp.float32),
        pltpu.VMEM((SLC1, EMB), jnp.float32),
        pltpu.VMEM((SLC0, EMB), jnp.float32),
        pltpu.VMEM((SLC1, EMB), jnp.float32),
        pltpu.VMEM((SLC0, EMB), jnp.float32),
        pltpu.VMEM((SLC1, EMB), jnp.float32),
        pltpu.VMEM((ROWS_PER_W, EMB), jnp.float32),
        pltpu.SemaphoreType.DMA,
        pltpu.SemaphoreType.DMA,
        pltpu.SemaphoreType.DMA,
        pltpu.SemaphoreType.DMA,
        pltpu.SemaphoreType.DMA,
        pltpu.SemaphoreType.DMA,
        pltpu.SemaphoreType.DMA,
        pltpu.SemaphoreType.DMA,
        pltpu.SemaphoreType.DMA,
        pltpu.SemaphoreType.DMA,
        pltpu.SemaphoreType.DMA,
        pltpu.SemaphoreType.DMA,
        pltpu.SemaphoreType.DMA,
        pltpu.SemaphoreType.DMA,
        pltpu.SemaphoreType.DMA,
        pltpu.SemaphoreType.DMA,
    ],
)(_pool_body)


def _mlp_body(pooled_ref, idx_ref, t0_ref, w1_ref, bb1_ref, w2_ref, bb2_ref,
              out_ref):
    pooled = pooled_ref[...]
    idx = idx_ref[...]
    c0 = jnp.sum((idx == 0).astype(jnp.float32), axis=1, keepdims=True)
    x = (pooled - c0 * t0_ref[...]) * (1.0 / SEQ)
    h = jnp.dot(x, w1_ref[...], preferred_element_type=jnp.float32,
                precision=lax.Precision.HIGHEST) + bb1_ref[...]
    z = jnp.dot(h, w2_ref[...], preferred_element_type=jnp.float32,
                precision=lax.Precision.HIGHEST) + bb2_ref[...]
    z = z - jnp.max(z, axis=1, keepdims=True)
    e = jnp.exp(z)
    out_ref[...] = e / jnp.sum(e, axis=1, keepdims=True)


def _mlp_call(pooled, idx, t0, w1, bb1, w2, bb2):
    bt = 512
    grid = (BATCH // bt,)
    return pl.pallas_call(
        _mlp_body,
        out_shape=jax.ShapeDtypeStruct((BATCH, OUT), jnp.float32),
        grid=grid,
        in_specs=[
            pl.BlockSpec((bt, EMB), lambda i: (i, 0)),
            pl.BlockSpec((bt, SEQ), lambda i: (i, 0)),
            pl.BlockSpec((1, EMB), lambda i: (0, 0)),
            pl.BlockSpec((EMB, HID), lambda i: (0, 0)),
            pl.BlockSpec((1, HID), lambda i: (0, 0)),
            pl.BlockSpec((HID, OUT), lambda i: (0, 0)),
            pl.BlockSpec((1, OUT), lambda i: (0, 0)),
        ],
        out_specs=pl.BlockSpec((bt, OUT), lambda i: (i, 0)),
    )(pooled, idx, t0, w1, bb1, w2, bb2)


def kernel(inputs, table, W1, b1, W2, b2):
    idx = inputs.astype(jnp.int32)
    pooled = _pooled_sum(table, idx)
    t0 = table[0:1]
    return _mlp_call(pooled, idx, t0, W1, b1.reshape(1, HID), W2,
                     b2.reshape(1, OUT))
